# baseline (device time: 42950 ns/iter reference)
import os

import jax
import jax.numpy as jnp
from jax import lax
from jax.experimental import pallas as pl
from jax.experimental.pallas import tpu as pltpu

_VARIANT = os.environ.get("KERNEL_VARIANT", "full")
_HAS_BARRIER = _VARIANT in ("barrier", "phaseA", "full")
_HAS_A = _VARIANT in ("phaseA", "full")
_HAS_B = _VARIANT == "full"

N_DEV = 32
N_TOK = 512
D_IN = 256
D_OUT = 512
N_EXP = 64
E_PER = 2
CH = N_TOK // N_DEV


def kernel(x, router_W, route_idx, expert_W, shared_W):
    def body(x_ref, rw_ref, idx_ref, ew_ref, sw_ref, out_ref,
             acc_ref, rs_ref, ssA, rsA, ssB, rsB):
        my = lax.axis_index("i")

        if _HAS_BARRIER:
            barrier_sem = pltpu.get_barrier_semaphore()
            for k in range(1, N_DEV):
                pl.semaphore_signal(barrier_sem, inc=1,
                                    device_id=(lax.rem(my + k, N_DEV),),
                                    device_id_type=pl.DeviceIdType.MESH)
            pl.semaphore_wait(barrier_sem, N_DEV - 1)

        x = x_ref[:, :]
        scores = jnp.dot(x, rw_ref[:, :], preferred_element_type=jnp.float32)
        m = jnp.max(scores, axis=-1, keepdims=True)
        e = jnp.exp(scores - m)
        probs = e / jnp.sum(e, axis=-1, keepdims=True)
        idx = idx_ref[:, :]
        col = lax.broadcasted_iota(jnp.int32, (N_TOK, N_EXP), 1)
        acc = None
        for k in range(E_PER):
            gl = my * E_PER + k
            p_gl = jnp.sum(jnp.where(col == gl, probs, 0.0), axis=-1,
                           keepdims=True)
            w = jnp.where(idx == gl, p_gl, 0.0)
            y = jnp.dot(x, ew_ref[k, :, :], preferred_element_type=jnp.float32)
            term = w * y
            acc = term if acc is None else acc + term
        acc_ref[:, :] = acc

        row_io = lax.broadcasted_iota(jnp.int32, (N_TOK, 1), 0)

        def blk_any(expert_dev, lo):
            mine = jnp.logical_or(idx == expert_dev * E_PER,
                                  idx == expert_dev * E_PER + 1)
            in_blk = jnp.logical_and(row_io >= lo, row_io < lo + CH)
            sel = jnp.where(jnp.logical_and(mine, in_blk), 1, 0)
            return jnp.sum(sel) > 0

        sendsA = []
        for k in range(1, N_DEV) if _HAS_A else ():
            tgt = lax.rem(my + k, N_DEV)
            has = blk_any(my, tgt * CH)
            rdma = pltpu.make_async_remote_copy(
                src_ref=acc_ref.at[pl.ds(tgt * CH, CH)],
                dst_ref=rs_ref.at[my],
                send_sem=ssA.at[k - 1],
                recv_sem=rsA.at[my],
                device_id=(tgt,),
                device_id_type=pl.DeviceIdType.MESH,
            )

            @pl.when(has)
            def _(rdma=rdma):
                rdma.start()

            sendsA.append((has, rdma))

        rs_ref[my, :, :] = acc_ref[pl.ds(my * CH, CH), :]
        sh_mine = jnp.dot(x_ref[pl.ds(my * CH, CH), :], sw_ref[:, :],
                          preferred_element_type=jnp.float32)

        for k in range(1, N_DEV) if _HAS_A else ():
            src = lax.rem(my + k, N_DEV)
            has = blk_any(src, my * CH)
            recv = pltpu.make_async_remote_copy(
                src_ref=rs_ref.at[src],
                dst_ref=rs_ref.at[src],
                send_sem=ssA.at[0],
                recv_sem=rsA.at[src],
                device_id=(src,),
                device_id_type=pl.DeviceIdType.MESH,
            )

            @pl.when(has)
            def _(recv=recv):
                recv.wait_recv()

            @pl.when(jnp.logical_not(has))
            def _(src=src):
                rs_ref[src, :, :] = jnp.zeros((CH, D_OUT), jnp.float32)

        reduced = jnp.sum(rs_ref[:, :, :], axis=0)
        out_ref[pl.ds(my * CH, CH), :] = reduced + sh_mine

        sendsB = []
        for k in range(1, N_DEV) if _HAS_B else ():
            tgt = lax.rem(my + k, N_DEV)
            rdma = pltpu.make_async_remote_copy(
                src_ref=out_ref.at[pl.ds(my * CH, CH)],
                dst_ref=out_ref.at[pl.ds(my * CH, CH)],
                send_sem=ssB.at[k - 1],
                recv_sem=rsB.at[my],
                device_id=(tgt,),
                device_id_type=pl.DeviceIdType.MESH,
            )
            rdma.start()
            sendsB.append(rdma)

        for has, rdma in sendsA:
            @pl.when(has)
            def _(rdma=rdma):
                rdma.wait_send()

        for k in range(1, N_DEV) if _HAS_B else ():
            src = lax.rem(my + k, N_DEV)
            recv = pltpu.make_async_remote_copy(
                src_ref=out_ref.at[pl.ds(src * CH, CH)],
                dst_ref=out_ref.at[pl.ds(src * CH, CH)],
                send_sem=ssB.at[0],
                recv_sem=rsB.at[src],
                device_id=(src,),
                device_id_type=pl.DeviceIdType.MESH,
            )
            recv.wait_recv()
        for rdma in sendsB:
            rdma.wait_send()

    return pl.pallas_call(
        body,
        out_shape=jax.ShapeDtypeStruct((N_TOK, D_OUT), jnp.float32),
        in_specs=[pl.BlockSpec(memory_space=pltpu.VMEM)] * 5,
        out_specs=pl.BlockSpec(memory_space=pltpu.VMEM),
        scratch_shapes=[
            pltpu.VMEM((N_TOK, D_OUT), jnp.float32),
            pltpu.VMEM((N_DEV, CH, D_OUT), jnp.float32),
            pltpu.SemaphoreType.DMA((N_DEV,)),
            pltpu.SemaphoreType.DMA((N_DEV,)),
            pltpu.SemaphoreType.DMA((N_DEV,)),
            pltpu.SemaphoreType.DMA((N_DEV,)),
        ],
        compiler_params=(pltpu.CompilerParams(collective_id=0)
                         if _HAS_BARRIER else pltpu.CompilerParams()),
    )(x, router_W, route_idx, expert_W, shared_W)
